# baseline (speedup 1.0000x reference)
import functools

import jax
import jax.numpy as jnp
from jax import lax
from jax.experimental import pallas as pl
from jax.experimental.pallas import tpu as pltpu
from jax.experimental.pallas import tpu_sc as plsc

B = 16384
D = 16
H = 512
NF = 8
E_COLS = 128

NC = 2
NS = 16
NW = NC * NS
BPW = B // NW
CHUNK = 128
NCH = BPW // CHUNK
NG = B // CHUNK

NSMALL = 3
NBIGC = 2 * D


def _sc_gather(idxs, idxb, t_scat, t_ecat, t_len, tT_sid, tT_eid):
  mesh = plsc.VectorSubcoreMesh(
      core_axis_name="c", subcore_axis_name="s",
      num_cores=NC, num_subcores=NS)

  @functools.partial(
      pl.kernel,
      out_type=(jax.ShapeDtypeStruct((B, E_COLS), jnp.float32),
                jax.ShapeDtypeStruct((NBIGC, NG, 1, CHUNK), jnp.float32)),
      mesh=mesh,
      compiler_params=pltpu.CompilerParams(use_tc_tiling_on_sc=False),
      scratch_types=[
          pltpu.VMEM((NSMALL, NCH, CHUNK), jnp.int32),
          pltpu.VMEM((NBIGC, NCH, CHUNK), jnp.int32),
          pltpu.VMEM((NSMALL, BPW, D), jnp.float32),
          pltpu.VMEM((NBIGC, NCH, 1, CHUNK), jnp.float32),
          pltpu.SemaphoreType.DMA,
      ],
  )
  def gather_kernel(idxs_hbm, idxb_hbm, tab0, tab1, tab2, tbig0, tbig1,
                    es_hbm, et_hbm, idxs_v, idxb_v, rows_v, bigt_v, sem):
    smalls = [tab0, tab1, tab2]
    bigs = [tbig0, tbig1]
    wid = lax.axis_index("s") * NC + lax.axis_index("c")
    g0 = wid * NCH
    base = wid * BPW
    pltpu.sync_copy(idxs_hbm.at[:, pl.ds(g0, NCH)], idxs_v)
    pltpu.sync_copy(idxb_hbm.at[:, pl.ds(g0, NCH)], idxb_v)
    copies = []
    for k in range(NSMALL):
      for c in range(NCH):
        copies.append(pltpu.async_copy(
            smalls[k].at[idxs_v.at[k, c]],
            rows_v.at[k, pl.ds(c * CHUNK, CHUNK)],
            sem))
    for jc in range(NBIGC):
      for c in range(NCH):
        copies.append(pltpu.async_copy(
            bigs[jc // D].at[idxb_v.at[jc, c]],
            bigt_v.at[jc, c, 0],
            sem))
    for cp in copies:
      cp.wait()
    for k in range(NSMALL):
      pltpu.sync_copy(rows_v.at[k],
                      es_hbm.at[pl.ds(base, BPW), pl.ds(k * D, D)])
    pltpu.sync_copy(bigt_v, et_hbm.at[:, pl.ds(g0, NCH)])

  return gather_kernel(idxs, idxb, t_scat, t_ecat, t_len, tT_sid, tT_eid)


MBT = 128


def _tc_dense_kernel(es_ref, et_ref, xt_ref, ws_ref, wb_ref, wx_ref, b_ref,
                     out_ref):
  acc = jnp.dot(es_ref[:, :NSMALL * D], ws_ref[...],
                preferred_element_type=jnp.float32)
  acc += lax.dot_general(et_ref[:, 0, 0, :], wb_ref[...],
                         (((0,), (0,)), ((), ())),
                         preferred_element_type=jnp.float32)
  acc += lax.dot_general(xt_ref[...], wx_ref[...],
                         (((0,), (0,)), ((), ())),
                         preferred_element_type=jnp.float32)
  out_ref[...] = acc + b_ref[...]


def _tc_dense(es, et, xt, ws, wb, wx, b2):
  return pl.pallas_call(
      _tc_dense_kernel,
      grid=(B // MBT,),
      in_specs=[
          pl.BlockSpec((MBT, E_COLS), lambda i: (i, 0)),
          pl.BlockSpec((NBIGC, 1, 1, CHUNK), lambda i: (0, i, 0, 0)),
          pl.BlockSpec((NF, MBT), lambda i: (0, i)),
          pl.BlockSpec((NSMALL * D, H), lambda i: (0, 0)),
          pl.BlockSpec((NBIGC, H), lambda i: (0, 0)),
          pl.BlockSpec((NF, H), lambda i: (0, 0)),
          pl.BlockSpec((1, H), lambda i: (0, 0)),
      ],
      out_specs=pl.BlockSpec((MBT, H), lambda i: (i, 0)),
      out_shape=jax.ShapeDtypeStruct((B, H), jnp.float32),
  )(es, et, xt, ws, wb, wx, b2)


def kernel(x, emb_sid, emb_scat, emb_eid, emb_ecat, emb_len, W, b):
  n_sid, n_eid = emb_sid.shape[0], emb_eid.shape[0]
  xi = x[:, 3:8].astype(jnp.int32)
  idxs = xi[:, (1, 3, 4)].T.reshape(NSMALL, NG, CHUNK)
  coff_sid = (jnp.arange(D, dtype=jnp.int32) * n_sid)[:, None]
  coff_eid = (jnp.arange(D, dtype=jnp.int32) * n_eid)[:, None]
  idxb = jnp.concatenate(
      [xi[:, 0][None, :] + coff_sid, xi[:, 2][None, :] + coff_eid],
      axis=0).reshape(NBIGC, NG, CHUNK)
  tT_sid = emb_sid.T.reshape(-1)
  tT_eid = emb_eid.T.reshape(-1)
  es, et = _sc_gather(idxs, idxb, emb_scat, emb_ecat, emb_len,
                      tT_sid, tT_eid)
  ws = jnp.concatenate([W[D:2 * D], W[3 * D:4 * D], W[4 * D:5 * D]], axis=0)
  wb = jnp.concatenate([W[0:D], W[2 * D:3 * D]], axis=0)
  wx = jnp.zeros((NF, H), jnp.float32).at[0:3].set(W[5 * D:])
  b2 = b.reshape(1, H)
  return _tc_dense(es, et, x.T, ws, wb, wx, b2)


# speedup vs baseline: 1.6251x; 1.6251x over previous
import functools

import jax
import jax.numpy as jnp
from jax import lax
from jax.experimental import pallas as pl
from jax.experimental.pallas import tpu as pltpu
from jax.experimental.pallas import tpu_sc as plsc

B = 16384
D = 16
H = 512
NF = 8
E_COLS = 128

NC = 2
NS = 16
NW = NC * NS
BPW = B // NW
CHUNK = 128
NCH = BPW // CHUNK
NG = B // CHUNK

NSMALL = 3
NBIGC = 2 * D


def _sc_gather(idxs, idxb, t_scat, t_ecat, t_len, tT_sid, tT_eid):
  mesh = plsc.VectorSubcoreMesh(
      core_axis_name="c", subcore_axis_name="s",
      num_cores=NC, num_subcores=NS)

  @functools.partial(
      pl.kernel,
      out_type=(jax.ShapeDtypeStruct((B, E_COLS), jnp.float32),
                jax.ShapeDtypeStruct((NBIGC, NG, 1, CHUNK), jnp.float32)),
      mesh=mesh,
      compiler_params=pltpu.CompilerParams(use_tc_tiling_on_sc=False),
      scratch_types=[
          pltpu.VMEM((NSMALL, NCH, CHUNK), jnp.int32),
          pltpu.VMEM((NBIGC, NCH, CHUNK), jnp.int32),
          pltpu.VMEM((NSMALL, BPW, D), jnp.float32),
          pltpu.VMEM((NBIGC, NCH, 1, CHUNK), jnp.float32),
          pltpu.SemaphoreType.DMA,
      ],
  )
  def gather_kernel(idxs_hbm, idxb_hbm, tab0, tab1, tab2, tbig0, tbig1,
                    es_hbm, et_hbm, idxs_v, idxb_v, rows_v, bigt_v, sem):
    smalls = [tab0, tab1, tab2]
    bigs = [tbig0, tbig1]
    wid = lax.axis_index("s") * NC + lax.axis_index("c")
    g0 = wid * NCH
    base = wid * BPW
    pltpu.sync_copy(idxs_hbm.at[:, pl.ds(g0, NCH)], idxs_v)
    pltpu.sync_copy(idxb_hbm.at[:, pl.ds(g0, NCH)], idxb_v)
    copies = []
    for k in range(NSMALL):
      for c in range(NCH):
        copies.append(pltpu.async_copy(
            smalls[k].at[idxs_v.at[k, c]],
            rows_v.at[k, pl.ds(c * CHUNK, CHUNK)],
            sem))
    for jc in range(NBIGC):
      for c in range(NCH):
        copies.append(pltpu.async_copy(
            bigs[jc // D].at[idxb_v.at[jc, c]],
            bigt_v.at[jc, c, 0],
            sem))
    for cp in copies:
      cp.wait()
    for k in range(NSMALL):
      pltpu.sync_copy(rows_v.at[k],
                      es_hbm.at[pl.ds(base, BPW), pl.ds(k * D, D)])
    pltpu.sync_copy(bigt_v, et_hbm.at[:, pl.ds(g0, NCH)])

  return gather_kernel(idxs, idxb, t_scat, t_ecat, t_len, tT_sid, tT_eid)


MB = 1024
SUB = MB // CHUNK


def _tc_dense_kernel(es_ref, et_ref, xt_ref, ws_ref, wb_ref, wx_ref, b_ref,
                     out_ref):
  for k in range(SUB):
    r = pl.ds(k * CHUNK, CHUNK)
    acc = jnp.dot(es_ref[r, :NSMALL * D], ws_ref[...],
                  preferred_element_type=jnp.float32)
    acc += lax.dot_general(et_ref[:, k, 0, :], wb_ref[...],
                           (((0,), (0,)), ((), ())),
                           preferred_element_type=jnp.float32)
    acc += lax.dot_general(xt_ref[:, r], wx_ref[...],
                           (((0,), (0,)), ((), ())),
                           preferred_element_type=jnp.float32)
    out_ref[r, :] = acc + b_ref[...]


def _tc_dense(es, et, xt, ws, wb, wx, b2):
  return pl.pallas_call(
      _tc_dense_kernel,
      grid=(B // MB,),
      in_specs=[
          pl.BlockSpec((MB, E_COLS), lambda i: (i, 0)),
          pl.BlockSpec((NBIGC, SUB, 1, CHUNK), lambda i: (0, i, 0, 0)),
          pl.BlockSpec((NF, MB), lambda i: (0, i)),
          pl.BlockSpec((NSMALL * D, H), lambda i: (0, 0)),
          pl.BlockSpec((NBIGC, H), lambda i: (0, 0)),
          pl.BlockSpec((NF, H), lambda i: (0, 0)),
          pl.BlockSpec((1, H), lambda i: (0, 0)),
      ],
      out_specs=pl.BlockSpec((MB, H), lambda i: (i, 0)),
      out_shape=jax.ShapeDtypeStruct((B, H), jnp.float32),
  )(es, et, xt, ws, wb, wx, b2)


def kernel(x, emb_sid, emb_scat, emb_eid, emb_ecat, emb_len, W, b):
  n_sid, n_eid = emb_sid.shape[0], emb_eid.shape[0]
  xi = x[:, 3:8].astype(jnp.int32)
  idxs = xi[:, (1, 3, 4)].T.reshape(NSMALL, NG, CHUNK)
  coff_sid = (jnp.arange(D, dtype=jnp.int32) * n_sid)[:, None]
  coff_eid = (jnp.arange(D, dtype=jnp.int32) * n_eid)[:, None]
  idxb = jnp.concatenate(
      [xi[:, 0][None, :] + coff_sid, xi[:, 2][None, :] + coff_eid],
      axis=0).reshape(NBIGC, NG, CHUNK)
  tT_sid = emb_sid.T.reshape(-1)
  tT_eid = emb_eid.T.reshape(-1)
  es, et = _sc_gather(idxs, idxb, emb_scat, emb_ecat, emb_len,
                      tT_sid, tT_eid)
  ws = jnp.concatenate([W[D:2 * D], W[3 * D:4 * D], W[4 * D:5 * D]], axis=0)
  wb = jnp.concatenate([W[0:D], W[2 * D:3 * D]], axis=0)
  wx = jnp.zeros((NF, H), jnp.float32).at[0:3].set(W[5 * D:])
  b2 = b.reshape(1, H)
  return _tc_dense(es, et, x.T, ws, wb, wx, b2)
